# baseline (speedup 1.0000x reference)
import dataclasses

import jax
import jax.numpy as jnp
from jax import lax
from jax.experimental import pallas as pl
from jax.experimental.pallas import tpu as pltpu
from jax.experimental.pallas import tpu_sc as plsc

C = 768
N = 65536
K = int(N * 0.6)
L = 16
NC = 2
NS = 16
NW = NC * NS
CPW = C // NW
HBINS = 256
REP = 8
HWORDS = HBINS * L
HTOT = HWORDS * REP
NQ = 4
QN = N // NQ

_mesh = plsc.VectorSubcoreMesh(core_axis_name="c", subcore_axis_name="s")

_cparams = pltpu.CompilerParams()
if "needs_layout_passes" in pltpu.CompilerParams.__dataclass_fields__:
    _cparams = dataclasses.replace(_cparams, needs_layout_passes=False)


def _make_sc_kernel():
    out_t = (
        jax.ShapeDtypeStruct((NW, 32), jnp.float32),
        jax.ShapeDtypeStruct((NW, 32), jnp.float32),
    )

    @jax.jit
    def run(x):
        @pl.kernel(
            out_type=out_t,
            mesh=_mesh,
            compiler_params=_cparams,
            scratch_types=[
                pltpu.VMEM((N,), jnp.float32),
                pltpu.VMEM((HTOT,), jnp.int32),
                pltpu.VMEM((32,), jnp.float32),
                pltpu.VMEM((32,), jnp.float32),
            ] + [pltpu.SemaphoreType.DMA] * NQ,
        )
        def sck(x_hbm, mn_hbm, mx_hbm, xv, hist, rmin, rmax, *sems):
            wid = lax.axis_index("s") * NC + lax.axis_index("c")
            lane = lax.broadcasted_iota(jnp.int32, (L,), 0)
            lanes = [lax.bitwise_or(lane, jnp.int32(u * HWORDS))
                     for u in range(REP)]
            ones = jnp.ones((L,), jnp.int32)
            zeros = jnp.zeros((L,), jnp.int32)
            zf = jnp.zeros((L,), jnp.float32)
            rmin[pl.ds(0, L)] = zf
            rmin[pl.ds(L, L)] = zf
            rmax[pl.ds(0, L)] = zf
            rmax[pl.ds(L, L)] = zf

            @plsc.parallel_loop(0, HTOT, step=L)
            def _(i):
                hist[pl.ds(i, L)] = zeros

            def scan_hist(kk):
                @plsc.parallel_loop(
                    0, HBINS,
                    carry=(jnp.int32(0), jnp.int32(0), jnp.int32(0)))
                def scan(g, carry):
                    cum, bkt, cbel = carry
                    acc = hist[pl.ds(g * L, L)]
                    hist[pl.ds(g * L, L)] = zeros
                    for u in range(1, REP):
                        off = u * HWORDS + g * L
                        acc = acc + hist[pl.ds(off, L)]
                        hist[pl.ds(off, L)] = zeros
                    s = jnp.sum(acc)
                    newcum = cum + s
                    take = jnp.logical_and(cum <= kk, newcum > kk)
                    bkt = jnp.where(take, g, bkt)
                    cbel = jnp.where(take, cum, cbel)
                    return newcum, bkt, cbel
                tot, bkt, cbel = scan
                return bkt, cbel, tot

            @pl.loop(0, CPW)
            def per_channel(j):
                ch = wid * CPW + j
                copies = [
                    pltpu.async_copy(
                        x_hbm.at[ch, pl.ds(q * QN, QN)],
                        xv.at[pl.ds(q * QN, QN)],
                        sems[q])
                    for q in range(NQ)
                ]

                runmin = jnp.full((L,), jnp.inf, jnp.float32)
                for q in range(NQ):
                    copies[q].wait()

                    @plsc.parallel_loop(q * QN, (q + 1) * QN, step=L * REP,
                                        carry=runmin)
                    def p1(i, rm):
                        for u in range(REP):
                            v = xv[pl.ds(i + u * L, L)]
                            iu = plsc.bitcast(v, jnp.int32)
                            a = lax.bitwise_and(iu, jnp.int32(0x7FFFFFFF))
                            rm = jnp.minimum(rm, plsc.bitcast(a, jnp.float32))
                            e = lax.shift_right_logical(a, 23)
                            idx = lax.bitwise_or(lax.shift_left(e, 4),
                                                 jnp.int32(u * HWORDS))
                            plsc.addupdate_scatter(hist, [idx], ones)
                        return rm
                    runmin = p1
                minval = jnp.min(runmin)

                ebkt, cbel1, tot1 = scan_hist(jnp.int32(K))
                minval = jnp.where(tot1 == N, minval, jnp.float32(1e30))

                base = jnp.full((L,), lax.shift_left(ebkt, 8), jnp.int32)
                lim = jnp.full((L,), jnp.uint32(HBINS))

                @plsc.parallel_loop(0, N, step=L * REP)
                def p2(i):
                    for u in range(REP):
                        v = xv[pl.ds(i + u * L, L)]
                        iu = plsc.bitcast(v, jnp.int32)
                        a = lax.bitwise_and(iu, jnp.int32(0x7FFFFFFF))
                        t = lax.shift_right_logical(a, 15) - base
                        mask = plsc.bitcast(t, jnp.uint32) < lim
                        idx = lax.bitwise_or(lax.shift_left(t, 4), lanes[u])
                        plsc.addupdate_scatter(hist, [idx], ones, mask=mask)

                mbkt, _, _ = scan_hist(jnp.int32(K) - cbel1)

                bits = lax.bitwise_or(
                    lax.shift_left(
                        lax.bitwise_or(lax.shift_left(ebkt, 8), mbkt), 15),
                    jnp.int32(0x4000))
                bitsv = jnp.full((L,), bits, jnp.int32)
                maxval = jnp.max(plsc.bitcast(bitsv, jnp.float32))

                lane0 = lane == 0
                jsplat = jnp.full((L,), j, jnp.int32)
                plsc.store_scatter(rmin, [jsplat], jnp.full((L,), minval),
                                   mask=lane0)
                plsc.store_scatter(rmax, [jsplat], jnp.full((L,), maxval),
                                   mask=lane0)

            pltpu.sync_copy(rmin, mn_hbm.at[wid])
            pltpu.sync_copy(rmax, mx_hbm.at[wid])

        return sck(x)

    return run


_sc_run = _make_sc_kernel()


def kernel(x):
    mn, mx = _sc_run(x)
    mn = mn[:, :CPW].reshape(C, 1)
    mx = mx[:, :CPW].reshape(C, 1)
    return mn, mx


# speedup vs baseline: 3.4733x; 3.4733x over previous
import dataclasses

import jax
import jax.numpy as jnp
from jax import lax
from jax.experimental import pallas as pl
from jax.experimental.pallas import tpu as pltpu
from jax.experimental.pallas import tpu_sc as plsc

C = 768
N = 65536
K = int(N * 0.6)
L = 16
NC = 2
NS = 16
NW = NC * NS
CPW = C // NW
BINS = 32768
SUP = 256
SPW = BINS // SUP // L
CH = 4096
NCHK = N // CH
RING = 4

_mesh = plsc.VectorSubcoreMesh(core_axis_name="c", subcore_axis_name="s")

_cparams = pltpu.CompilerParams()
if "needs_layout_passes" in pltpu.CompilerParams.__dataclass_fields__:
    _cparams = dataclasses.replace(_cparams, needs_layout_passes=False)


def _make_sc_kernel():
    out_t = (
        jax.ShapeDtypeStruct((NW, 32), jnp.float32),
        jax.ShapeDtypeStruct((NW, 32), jnp.float32),
    )

    @jax.jit
    def run(x):
        @pl.kernel(
            out_type=out_t,
            mesh=_mesh,
            compiler_params=_cparams,
            scratch_types=[
                pltpu.VMEM((RING * CH,), jnp.float32),
                pltpu.VMEM((BINS,), jnp.int32),
                pltpu.VMEM((32,), jnp.float32),
                pltpu.VMEM((32,), jnp.float32),
            ] + [pltpu.SemaphoreType.DMA] * RING,
        )
        def sck(x_hbm, mn_hbm, mx_hbm, ring, hist, rmin, rmax, *sems):
            wid = lax.axis_index("s") * NC + lax.axis_index("c")
            lane = lax.broadcasted_iota(jnp.int32, (L,), 0)
            zeros = jnp.zeros((L,), jnp.int32)
            ones = jnp.ones((L,), jnp.int32)
            zf = jnp.zeros((L,), jnp.float32)
            rmin[pl.ds(0, L)] = zf
            rmin[pl.ds(L, L)] = zf
            rmax[pl.ds(0, L)] = zf
            rmax[pl.ds(L, L)] = zf

            @plsc.parallel_loop(0, BINS, step=L, unroll=8)
            def _(i):
                hist[pl.ds(i, L)] = zeros

            @pl.loop(0, CPW)
            def per_channel(j):
                ch = wid * CPW + j

                def issue(c):
                    slot = c % RING
                    return pltpu.async_copy(
                        x_hbm.at[ch, pl.ds(c * CH, CH)],
                        ring.at[pl.ds(slot * CH, CH)],
                        sems[slot])

                handles = {}
                for c in range(RING - 1):
                    handles[c] = issue(c)

                runmin = jnp.full((L,), jnp.int32(0x7FFFFFFF))
                for c in range(NCHK):
                    handles.pop(c).wait()
                    base = (c % RING) * CH

                    @plsc.parallel_loop(base, base + CH, step=L, unroll=8,
                                        carry=runmin)
                    def sweep(i, rm):
                        v = ring[pl.ds(i, L)]
                        a = lax.bitwise_and(plsc.bitcast(v, jnp.int32),
                                            jnp.int32(0x7FFFFFFF))
                        rm = jnp.minimum(rm, a)
                        w = lax.shift_right_logical(a, 16)
                        plsc.addupdate_scatter(hist, [w], ones)
                        return rm
                    runmin = sweep
                    if c + RING - 1 < NCHK:
                        handles[c + RING - 1] = issue(c + RING - 1)

                mini = jnp.min(runmin)

                cap0 = tuple(zeros for _ in range(SPW))

                @plsc.parallel_loop(
                    0, SUP,
                    carry=(jnp.int32(0), jnp.int32(0), jnp.int32(0)) + cap0)
                def scana(p, carry):
                    cum, sp, cbel = carry[:3]
                    cap = carry[3:]
                    hv = []
                    acc = None
                    for t in range(SPW):
                        h = hist[pl.ds(p * (SPW * L) + t * L, L)]
                        hist[pl.ds(p * (SPW * L) + t * L, L)] = zeros
                        hv.append(h)
                        acc = h if acc is None else acc + h
                    s = jnp.sum(acc)
                    newcum = cum + s
                    take = jnp.logical_and(cum <= K, newcum > K)
                    sp = jnp.where(take, p, sp)
                    cbel = jnp.where(take, cum, cbel)
                    cap = tuple(jnp.where(take, hv[t], cap[t])
                                for t in range(SPW))
                    return (newcum, sp, cbel) + cap

                tot, sp, cbela = scana[:3]
                cap = scana[3:]

                kk2 = jnp.int32(K) - cbela
                cumb = jnp.int32(0)
                gsel = jnp.int32(0)
                cbelb = jnp.int32(0)
                hsel = zeros
                for t in range(SPW):
                    s = jnp.sum(cap[t])
                    newcum = cumb + s
                    take = jnp.logical_and(cumb <= kk2, newcum > kk2)
                    gsel = jnp.where(take, t, gsel)
                    cbelb = jnp.where(take, cumb, cbelb)
                    hsel = jnp.where(take, cap[t], hsel)
                    cumb = newcum
                kk3 = kk2 - cbelb
                cml = jnp.cumsum(hsel)
                pos = jnp.sum(jnp.where(cml <= kk3, ones, zeros))

                wstar = (lax.shift_left(sp, 7) | lax.shift_left(gsel, 4)
                         | pos)
                bits = lax.bitwise_or(lax.shift_left(wstar, 16),
                                      jnp.int32(0x8000))
                mini = jnp.where(tot == N, mini, jnp.int32(0x7F000000))

                minv = jnp.max(plsc.bitcast(jnp.full((L,), mini), jnp.float32))
                maxv = jnp.max(plsc.bitcast(jnp.full((L,), bits), jnp.float32))

                lane0 = lane == 0
                jsplat = jnp.full((L,), j, jnp.int32)
                plsc.store_scatter(rmin, [jsplat], jnp.full((L,), minv),
                                   mask=lane0)
                plsc.store_scatter(rmax, [jsplat], jnp.full((L,), maxv),
                                   mask=lane0)

            pltpu.sync_copy(rmin, mn_hbm.at[wid])
            pltpu.sync_copy(rmax, mx_hbm.at[wid])

        return sck(x)

    return run


_sc_run = _make_sc_kernel()


def kernel(x):
    mn, mx = _sc_run(x)
    mn = mn[:, :CPW].reshape(C, 1)
    mx = mx[:, :CPW].reshape(C, 1)
    return mn, mx


# speedup vs baseline: 4.1228x; 1.1870x over previous
import dataclasses

import jax
import jax.numpy as jnp
from jax import lax
from jax.experimental import pallas as pl
from jax.experimental.pallas import tpu as pltpu
from jax.experimental.pallas import tpu_sc as plsc

C = 768
N = 65536
K = int(N * 0.6)
L = 16
NC = 2
NS = 16
NW = NC * NS
CPW = C // NW
BINS = 16384
SUP = 256
SPW = BINS // SUP // L
CH = 4096
NCHK = N // CH
RING = 4

_mesh = plsc.VectorSubcoreMesh(core_axis_name="c", subcore_axis_name="s")

_cparams = pltpu.CompilerParams()
if "needs_layout_passes" in pltpu.CompilerParams.__dataclass_fields__:
    _cparams = dataclasses.replace(_cparams, needs_layout_passes=False)


def _make_sc_kernel():
    out_t = (
        jax.ShapeDtypeStruct((NW, 32), jnp.float32),
        jax.ShapeDtypeStruct((NW, 32), jnp.float32),
    )

    @jax.jit
    def run(x):
        @pl.kernel(
            out_type=out_t,
            mesh=_mesh,
            compiler_params=_cparams,
            scratch_types=[
                pltpu.VMEM((RING * CH,), jnp.float32),
                pltpu.VMEM((BINS,), jnp.int32),
                pltpu.VMEM((32,), jnp.float32),
                pltpu.VMEM((32,), jnp.float32),
            ] + [pltpu.SemaphoreType.DMA] * RING,
        )
        def sck(x_hbm, mn_hbm, mx_hbm, ring, hist, rmin, rmax, *sems):
            wid = lax.axis_index("s") * NC + lax.axis_index("c")
            lane = lax.broadcasted_iota(jnp.int32, (L,), 0)
            zeros = jnp.zeros((L,), jnp.int32)
            ones = jnp.ones((L,), jnp.int32)
            zf = jnp.zeros((L,), jnp.float32)
            rmin[pl.ds(0, L)] = zf
            rmin[pl.ds(L, L)] = zf
            rmax[pl.ds(0, L)] = zf
            rmax[pl.ds(L, L)] = zf

            @plsc.parallel_loop(0, BINS, step=L, unroll=8)
            def _(i):
                hist[pl.ds(i, L)] = zeros

            def issue(chrow, c):
                pltpu.async_copy(
                    x_hbm.at[chrow, pl.ds(c * CH, CH)],
                    ring.at[pl.ds((c % RING) * CH, CH)],
                    sems[c % RING])

            def wait_slot(slot):
                pltpu.make_async_copy(
                    x_hbm.at[0, pl.ds(0, CH)],
                    ring.at[pl.ds(slot * CH, CH)],
                    sems[slot]).wait()

            for c in range(RING - 1):
                issue(wid * CPW, c)

            @pl.loop(0, CPW)
            def per_channel(j):
                ch = wid * CPW + j
                chnext = jnp.minimum(ch + 1, C - 1)

                runmin = jnp.full((L,), jnp.uint32(0xFFFFFFFF))
                for c in range(NCHK):
                    wait_slot(c % RING)
                    base = (c % RING) * CH

                    @plsc.parallel_loop(base, base + CH, step=L, unroll=8,
                                        carry=runmin)
                    def sweep(i, rm):
                        v = ring[pl.ds(i, L)]
                        s = lax.shift_left(plsc.bitcast(v, jnp.uint32),
                                           jnp.uint32(1))
                        rm = jnp.minimum(rm, s)
                        w = lax.shift_right_logical(s, jnp.uint32(18))
                        plsc.addupdate_scatter(hist,
                                               [plsc.bitcast(w, jnp.int32)],
                                               ones)
                        return rm
                    runmin = sweep
                    nc = c + RING - 1
                    if nc < NCHK:
                        issue(ch, nc)
                    else:
                        issue(chnext, nc - NCHK)

                rm_s = plsc.bitcast(runmin, jnp.int32) ^ jnp.int32(-2**31)
                mini = (jnp.min(rm_s) ^ jnp.int32(-2**31))
                mini = lax.shift_right_logical(mini, 1)

                cap0 = tuple(zeros for _ in range(SPW))

                @plsc.parallel_loop(
                    0, SUP,
                    carry=(jnp.int32(0), jnp.int32(0), jnp.int32(0)) + cap0)
                def scana(p, carry):
                    cum, sp, cbel = carry[:3]
                    cap = carry[3:]
                    hv = []
                    acc = None
                    for t in range(SPW):
                        h = hist[pl.ds(p * (SPW * L) + t * L, L)]
                        hist[pl.ds(p * (SPW * L) + t * L, L)] = zeros
                        hv.append(h)
                        acc = h if acc is None else acc + h
                    s = jnp.sum(acc)
                    newcum = cum + s
                    take = jnp.logical_and(cum <= K, newcum > K)
                    sp = jnp.where(take, p, sp)
                    cbel = jnp.where(take, cum, cbel)
                    cap = tuple(jnp.where(take, hv[t], cap[t])
                                for t in range(SPW))
                    return (newcum, sp, cbel) + cap

                tot, sp, cbela = scana[:3]
                cap = scana[3:]

                kk2 = jnp.int32(K) - cbela
                cumb = jnp.int32(0)
                gsel = jnp.int32(0)
                cbelb = jnp.int32(0)
                hsel = zeros
                for t in range(SPW):
                    s = jnp.sum(cap[t])
                    newcum = cumb + s
                    take = jnp.logical_and(cumb <= kk2, newcum > kk2)
                    gsel = jnp.where(take, t, gsel)
                    cbelb = jnp.where(take, cumb, cbelb)
                    hsel = jnp.where(take, cap[t], hsel)
                    cumb = newcum
                kk3 = kk2 - cbelb
                cml = jnp.cumsum(hsel)
                pos = jnp.sum(jnp.where(cml <= kk3, ones, zeros))

                wstar = (lax.shift_left(sp, 6) | lax.shift_left(gsel, 4)
                         | pos)
                bits = lax.bitwise_or(lax.shift_left(wstar, 17),
                                      jnp.int32(0x10000))
                mini = jnp.where(tot == N, mini, jnp.int32(0x7F000000))

                minv = jnp.max(plsc.bitcast(jnp.full((L,), mini), jnp.float32))
                maxv = jnp.max(plsc.bitcast(jnp.full((L,), bits), jnp.float32))

                lane0 = lane == 0
                jsplat = jnp.full((L,), j, jnp.int32)
                plsc.store_scatter(rmin, [jsplat], jnp.full((L,), minv),
                                   mask=lane0)
                plsc.store_scatter(rmax, [jsplat], jnp.full((L,), maxv),
                                   mask=lane0)

            for slot in range(RING - 1):
                wait_slot(slot)

            pltpu.sync_copy(rmin, mn_hbm.at[wid])
            pltpu.sync_copy(rmax, mx_hbm.at[wid])

        return sck(x)

    return run


_sc_run = _make_sc_kernel()


def kernel(x):
    mn, mx = _sc_run(x)
    mn = mn[:, :CPW].reshape(C, 1)
    mx = mx[:, :CPW].reshape(C, 1)
    return mn, mx


# speedup vs baseline: 4.6998x; 1.1400x over previous
import dataclasses

import jax
import jax.numpy as jnp
from jax import lax
from jax.experimental import pallas as pl
from jax.experimental.pallas import tpu as pltpu
from jax.experimental.pallas import tpu_sc as plsc

C = 768
N = 65536
K = int(N * 0.6)
L = 16
NC = 2
NS = 16
NW = NC * NS
CPW = C // NW
BINS = 16384
SUP = 256
SPW = BINS // SUP // L
CH = 8192
NCHK = N // CH
RING = 4

_mesh = plsc.VectorSubcoreMesh(core_axis_name="c", subcore_axis_name="s")

_cparams = pltpu.CompilerParams()
if "needs_layout_passes" in pltpu.CompilerParams.__dataclass_fields__:
    _cparams = dataclasses.replace(_cparams, needs_layout_passes=False)


def _make_sc_kernel():
    out_t = (
        jax.ShapeDtypeStruct((NW, 32), jnp.float32),
        jax.ShapeDtypeStruct((NW, 32), jnp.float32),
    )

    @jax.jit
    def run(x):
        @pl.kernel(
            out_type=out_t,
            mesh=_mesh,
            compiler_params=_cparams,
            scratch_types=[
                pltpu.VMEM((RING * CH,), jnp.float32),
                pltpu.VMEM((BINS,), jnp.int32),
                pltpu.VMEM((32,), jnp.float32),
                pltpu.VMEM((32,), jnp.float32),
            ] + [pltpu.SemaphoreType.DMA] * RING,
        )
        def sck(x_hbm, mn_hbm, mx_hbm, ring, hist, rmin, rmax, *sems):
            wid = lax.axis_index("s") * NC + lax.axis_index("c")
            lane = lax.broadcasted_iota(jnp.int32, (L,), 0)
            zeros = jnp.zeros((L,), jnp.int32)
            ones = jnp.ones((L,), jnp.int32)
            zf = jnp.zeros((L,), jnp.float32)
            rmin[pl.ds(0, L)] = zf
            rmin[pl.ds(L, L)] = zf
            rmax[pl.ds(0, L)] = zf
            rmax[pl.ds(L, L)] = zf

            @plsc.parallel_loop(0, BINS, step=L, unroll=8)
            def _(i):
                hist[pl.ds(i, L)] = zeros

            def issue(chrow, c):
                pltpu.async_copy(
                    x_hbm.at[chrow, pl.ds(c * CH, CH)],
                    ring.at[pl.ds((c % RING) * CH, CH)],
                    sems[c % RING])

            def wait_slot(slot):
                pltpu.make_async_copy(
                    x_hbm.at[0, pl.ds(0, CH)],
                    ring.at[pl.ds(slot * CH, CH)],
                    sems[slot]).wait()

            for c in range(RING - 1):
                issue(wid * CPW, c)

            @pl.loop(0, CPW)
            def per_channel(j):
                ch = wid * CPW + j
                chnext = jnp.minimum(ch + 1, C - 1)

                runmin = jnp.full((L,), jnp.uint32(0xFFFFFFFF))
                for c in range(NCHK):
                    wait_slot(c % RING)
                    base = (c % RING) * CH

                    @plsc.parallel_loop(base, base + CH, step=L, unroll=8,
                                        carry=runmin)
                    def sweep(i, rm):
                        v = ring[pl.ds(i, L)]
                        s = lax.shift_left(plsc.bitcast(v, jnp.uint32),
                                           jnp.uint32(1))
                        rm = jnp.minimum(rm, s)
                        w = lax.shift_right_logical(s, jnp.uint32(18))
                        plsc.addupdate_scatter(hist,
                                               [plsc.bitcast(w, jnp.int32)],
                                               ones)
                        return rm
                    runmin = sweep
                    nc = c + RING - 1
                    if nc < NCHK:
                        issue(ch, nc)
                    else:
                        issue(chnext, nc - NCHK)

                rm_s = plsc.bitcast(runmin, jnp.int32) ^ jnp.int32(-2**31)
                mini = (jnp.min(rm_s) ^ jnp.int32(-2**31))
                mini = lax.shift_right_logical(mini, 1)

                cap0 = tuple(zeros for _ in range(SPW))

                @plsc.parallel_loop(
                    0, SUP,
                    carry=(jnp.int32(0), jnp.int32(0), jnp.int32(0)) + cap0,
                    unroll=2)
                def scana(p, carry):
                    cum, sp, cbel = carry[:3]
                    cap = carry[3:]
                    hv = []
                    acc = None
                    for t in range(SPW):
                        h = hist[pl.ds(p * (SPW * L) + t * L, L)]
                        hist[pl.ds(p * (SPW * L) + t * L, L)] = zeros
                        hv.append(h)
                        acc = h if acc is None else acc + h
                    s = jnp.sum(acc)
                    newcum = cum + s
                    take = jnp.logical_and(cum <= K, newcum > K)
                    sp = jnp.where(take, p, sp)
                    cbel = jnp.where(take, cum, cbel)
                    cap = tuple(jnp.where(take, hv[t], cap[t])
                                for t in range(SPW))
                    return (newcum, sp, cbel) + cap

                tot, sp, cbela = scana[:3]
                cap = scana[3:]

                kk2 = jnp.int32(K) - cbela
                cumb = jnp.int32(0)
                gsel = jnp.int32(0)
                cbelb = jnp.int32(0)
                hsel = zeros
                for t in range(SPW):
                    s = jnp.sum(cap[t])
                    newcum = cumb + s
                    take = jnp.logical_and(cumb <= kk2, newcum > kk2)
                    gsel = jnp.where(take, t, gsel)
                    cbelb = jnp.where(take, cumb, cbelb)
                    hsel = jnp.where(take, cap[t], hsel)
                    cumb = newcum
                kk3 = kk2 - cbelb
                cml = jnp.cumsum(hsel)
                pos = jnp.sum(jnp.where(cml <= kk3, ones, zeros))

                wstar = (lax.shift_left(sp, 6) | lax.shift_left(gsel, 4)
                         | pos)
                bits = lax.bitwise_or(lax.shift_left(wstar, 17),
                                      jnp.int32(0x10000))
                mini = jnp.where(tot == N, mini, jnp.int32(0x7F000000))

                minv = jnp.max(plsc.bitcast(jnp.full((L,), mini), jnp.float32))
                maxv = jnp.max(plsc.bitcast(jnp.full((L,), bits), jnp.float32))

                lane0 = lane == 0
                jsplat = jnp.full((L,), j, jnp.int32)
                plsc.store_scatter(rmin, [jsplat], jnp.full((L,), minv),
                                   mask=lane0)
                plsc.store_scatter(rmax, [jsplat], jnp.full((L,), maxv),
                                   mask=lane0)

            for slot in range(RING - 1):
                wait_slot(slot)

            pltpu.sync_copy(rmin, mn_hbm.at[wid])
            pltpu.sync_copy(rmax, mx_hbm.at[wid])

        return sck(x)

    return run


_sc_run = _make_sc_kernel()


def kernel(x):
    mn, mx = _sc_run(x)
    mn = mn[:, :CPW].reshape(C, 1)
    mx = mx[:, :CPW].reshape(C, 1)
    return mn, mx
